# Initial kernel scaffold; baseline (speedup 1.0000x reference)
import functools

import jax
import jax.numpy as jnp
from jax import lax
from jax.experimental import pallas as pl
from jax.experimental.pallas import tpu as pltpu
from jax.experimental.pallas import tpu_sc as plsc

N_NODES = 10000
D_FEAT = 128
N_EDGES = 320000

NUM_CORES = 2
NUM_SUBCORES = 16
NUM_WORKERS = NUM_CORES * NUM_SUBCORES

CHUNK = 128
CHUNKS_PER_WORKER = -(-N_EDGES // (NUM_WORKERS * CHUNK))
EDGES_PER_WORKER = CHUNKS_PER_WORKER * CHUNK
E_PAD = EDGES_PER_WORKER * NUM_WORKERS

ACC_ROWS = 10240
TRASH_ROW = N_NODES
ROWS_PER_TILE = ACC_ROWS // NUM_SUBCORES


def _sc_agg_body(x_hbm, src_hbm, dst_hbm, part_hbm, acc, src_v, dst_v, rows_v, sem):
    c = lax.axis_index("c")
    s = lax.axis_index("s")
    w = c * NUM_SUBCORES + s

    zeros16 = jnp.zeros((16,), jnp.float32)
    rows_flat = rows_v.reshape(CHUNK * D_FEAT)

    def _zero_body(i, _):
        rows_flat[pl.ds(i * 16, 16)] = zeros16
        return 0

    lax.fori_loop(0, CHUNK * D_FEAT // 16, _zero_body, 0)
    for k in range(ROWS_PER_TILE // CHUNK):
        pltpu.sync_copy(rows_v, acc.at[pl.ds(s * ROWS_PER_TILE + k * CHUNK, CHUNK)])
    plsc.subcore_barrier()

    base = w * EDGES_PER_WORKER

    def _edge_body(j, _):
        off = base + j * CHUNK
        pltpu.sync_copy(src_hbm.at[pl.ds(off, CHUNK)], src_v)
        pltpu.sync_copy(dst_hbm.at[pl.ds(off, CHUNK)], dst_v)
        pltpu.async_copy(x_hbm.at[src_v], rows_v, sem).wait()
        pltpu.sync_copy(rows_v, acc.at[dst_v], add=True)
        return 0

    lax.fori_loop(0, CHUNKS_PER_WORKER, _edge_body, 0)
    plsc.subcore_barrier()

    pltpu.sync_copy(
        acc.at[pl.ds(s * ROWS_PER_TILE, ROWS_PER_TILE)],
        part_hbm.at[c, pl.ds(s * ROWS_PER_TILE, ROWS_PER_TILE)],
    )


@jax.jit
def _sc_aggregate(x, src, dst):
    mesh = plsc.VectorSubcoreMesh(core_axis_name="c", subcore_axis_name="s")
    return pl.kernel(
        _sc_agg_body,
        out_type=jax.ShapeDtypeStruct((NUM_CORES, ACC_ROWS, D_FEAT), jnp.float32),
        mesh=mesh,
        scratch_types=[
            pltpu.VMEM_SHARED((ACC_ROWS, D_FEAT), jnp.float32),
            pltpu.VMEM((CHUNK,), jnp.int32),
            pltpu.VMEM((CHUNK,), jnp.int32),
            pltpu.VMEM((CHUNK, D_FEAT), jnp.float32),
            pltpu.SemaphoreType.DMA,
        ],
    )(x, src, dst)


def _mlp_body(part_ref, x_ref, scale_ref, w1_ref, b1_ref, w2_ref, b2_ref, out_ref):
    scale = scale_ref[0, 0]
    y = part_ref[0] + part_ref[1] + scale * x_ref[...]
    h = jnp.dot(y, w1_ref[...], preferred_element_type=jnp.float32) + b1_ref[...]
    h = jnp.maximum(h, 0.0)
    out_ref[...] = jnp.dot(h, w2_ref[...], preferred_element_type=jnp.float32) + b2_ref[...]


@jax.jit
def _mlp(part, x, scale, W1, b1, W2, b2):
    br = 1000
    grid = (N_NODES // br,)
    return pl.pallas_call(
        _mlp_body,
        grid=grid,
        in_specs=[
            pl.BlockSpec((NUM_CORES, br, D_FEAT), lambda i: (0, i, 0)),
            pl.BlockSpec((br, D_FEAT), lambda i: (i, 0)),
            pl.BlockSpec(memory_space=pltpu.SMEM),
            pl.BlockSpec((D_FEAT, D_FEAT), lambda i: (0, 0)),
            pl.BlockSpec((1, D_FEAT), lambda i: (0, 0)),
            pl.BlockSpec((D_FEAT, D_FEAT), lambda i: (0, 0)),
            pl.BlockSpec((1, D_FEAT), lambda i: (0, 0)),
        ],
        out_specs=pl.BlockSpec((br, D_FEAT), lambda i: (i, 0)),
        out_shape=jax.ShapeDtypeStruct((N_NODES, D_FEAT), jnp.float32),
    )(part, x, scale, W1, b1, W2, b2)


def kernel(x, edge_index, eps, W1, b1, W2, b2):
    src = edge_index[0].astype(jnp.int32)
    dst = edge_index[1].astype(jnp.int32)
    pad = E_PAD - N_EDGES
    src = jnp.concatenate([src, jnp.zeros((pad,), jnp.int32)])
    dst = jnp.concatenate([dst, jnp.full((pad,), TRASH_ROW, jnp.int32)])
    part = _sc_aggregate(x, src, dst)
    scale = (1.0 + eps).reshape(1, 1)
    return _mlp(part, x, scale, W1, b1.reshape(1, D_FEAT), W2, b2.reshape(1, D_FEAT))


# speedup vs baseline: 4.4696x; 4.4696x over previous
import functools

import jax
import jax.numpy as jnp
from jax import lax
from jax.experimental import pallas as pl
from jax.experimental.pallas import tpu as pltpu
from jax.experimental.pallas import tpu_sc as plsc

N_NODES = 10000
D_FEAT = 128
N_EDGES = 320000

NUM_CORES = 2
NUM_SUBCORES = 16
NUM_WORKERS = NUM_CORES * NUM_SUBCORES

CHUNK = 128
CHUNKS_PER_WORKER = -(-N_EDGES // (NUM_WORKERS * CHUNK))
EDGES_PER_WORKER = CHUNKS_PER_WORKER * CHUNK
E_PAD = EDGES_PER_WORKER * NUM_WORKERS

ACC_ROWS = 10240
TRASH_ROW = N_NODES
ROWS_PER_TILE = ACC_ROWS // NUM_SUBCORES


def _sc_agg_body(x_hbm, src_hbm, dst_hbm, part_hbm, acc, src_v, dst_v, rows_v, sem):
    c = lax.axis_index("c")
    s = lax.axis_index("s")
    w = c * NUM_SUBCORES + s

    zeros16 = jnp.zeros((16,), jnp.float32)

    def _zero_body(r, _):
        for col in range(D_FEAT // 16):
            rows_v[r, pl.ds(col * 16, 16)] = zeros16
        return 0

    lax.fori_loop(0, CHUNK, _zero_body, 0)
    for k in range(ROWS_PER_TILE // CHUNK):
        pltpu.sync_copy(rows_v, acc.at[pl.ds(s * ROWS_PER_TILE + k * CHUNK, CHUNK)])
    plsc.subcore_barrier()

    base = w * EDGES_PER_WORKER

    def _edge_body(j, _):
        off = base + j * CHUNK
        pltpu.sync_copy(src_hbm.at[pl.ds(off, CHUNK)], src_v)
        pltpu.sync_copy(dst_hbm.at[pl.ds(off, CHUNK)], dst_v)
        pltpu.async_copy(x_hbm.at[src_v], rows_v, sem).wait()
        pltpu.sync_copy(rows_v, acc.at[dst_v], add=True)
        return 0

    lax.fori_loop(0, CHUNKS_PER_WORKER, _edge_body, 0)
    plsc.subcore_barrier()

    pltpu.sync_copy(
        acc.at[pl.ds(s * ROWS_PER_TILE, ROWS_PER_TILE)],
        part_hbm.at[c, pl.ds(s * ROWS_PER_TILE, ROWS_PER_TILE)],
    )


@jax.jit
def _sc_aggregate(x, src, dst):
    mesh = plsc.VectorSubcoreMesh(core_axis_name="c", subcore_axis_name="s")
    return pl.kernel(
        _sc_agg_body,
        out_type=jax.ShapeDtypeStruct((NUM_CORES, ACC_ROWS, D_FEAT), jnp.float32),
        mesh=mesh,
        scratch_types=[
            pltpu.VMEM_SHARED((ACC_ROWS, D_FEAT), jnp.float32),
            pltpu.VMEM((CHUNK,), jnp.int32),
            pltpu.VMEM((CHUNK,), jnp.int32),
            pltpu.VMEM((CHUNK, D_FEAT), jnp.float32),
            pltpu.SemaphoreType.DMA,
        ],
    )(x, src, dst)


def _mlp_body(part_ref, x_ref, scale_ref, w1_ref, b1_ref, w2_ref, b2_ref, out_ref):
    scale = scale_ref[0, 0]
    y = part_ref[0] + part_ref[1] + scale * x_ref[...]
    h = jnp.dot(y, w1_ref[...], preferred_element_type=jnp.float32) + b1_ref[...]
    h = jnp.maximum(h, 0.0)
    out_ref[...] = jnp.dot(h, w2_ref[...], preferred_element_type=jnp.float32) + b2_ref[...]


@jax.jit
def _mlp(part, x, scale, W1, b1, W2, b2):
    br = 1000
    grid = (N_NODES // br,)
    return pl.pallas_call(
        _mlp_body,
        grid=grid,
        in_specs=[
            pl.BlockSpec((NUM_CORES, br, D_FEAT), lambda i: (0, i, 0)),
            pl.BlockSpec((br, D_FEAT), lambda i: (i, 0)),
            pl.BlockSpec(memory_space=pltpu.SMEM),
            pl.BlockSpec((D_FEAT, D_FEAT), lambda i: (0, 0)),
            pl.BlockSpec((1, D_FEAT), lambda i: (0, 0)),
            pl.BlockSpec((D_FEAT, D_FEAT), lambda i: (0, 0)),
            pl.BlockSpec((1, D_FEAT), lambda i: (0, 0)),
        ],
        out_specs=pl.BlockSpec((br, D_FEAT), lambda i: (i, 0)),
        out_shape=jax.ShapeDtypeStruct((N_NODES, D_FEAT), jnp.float32),
    )(part, x, scale, W1, b1, W2, b2)


def kernel(x, edge_index, eps, W1, b1, W2, b2):
    src = edge_index[0].astype(jnp.int32)
    dst = edge_index[1].astype(jnp.int32)
    pad = E_PAD - N_EDGES
    src = jnp.concatenate([src, jnp.zeros((pad,), jnp.int32)])
    dst = jnp.concatenate([dst, jnp.full((pad,), TRASH_ROW, jnp.int32)])
    part = _sc_aggregate(x, src, dst)
    scale = (1.0 + eps).reshape(1, 1)
    return _mlp(part, x, scale, W1, b1.reshape(1, D_FEAT), W2, b2.reshape(1, D_FEAT))
